# Initial kernel scaffold; baseline (speedup 1.0000x reference)
import jax
import jax.numpy as jnp
from jax.experimental import pallas as pl


def kernel(x, edge_index, edge_attr, batch, node_embedding, edge_W, edge_b, msg_W1, msg_b1, msg_W2, msg_b2, soft_W, soft_b, upd_W1, upd_b1, upd_W2, upd_b2, nout_W1, nout_b1, nout_W2, nout_b2, ro_W1, ro_b1, ro_W2, ro_b2):
    raise NotImplementedError("write your pallas kernel here")


# speedup vs baseline: 2.7261x; 2.7261x over previous
import functools

import jax
import jax.numpy as jnp
import numpy as np
from jax import lax
from jax.experimental import pallas as pl
from jax.experimental.pallas import tpu as pltpu
from jax.experimental.pallas import tpu_sc as plsc

F32 = jnp.float32

N = 10000
E = 320000
H = 128
G = 512
TDIM = 32
DEPTH = 4
NENC = 4

N_PAD = 10240
BN = 2048
BE = 3200

NC, NS = 2, 16
NW = NC * NS
E_PER_W = E // NW
EC = 128
NFULL = E_PER_W // EC
TAIL = E_PER_W - NFULL * EC
ROWS_PER_TILE = N_PAD // NS


def _silu(v):
    return v * jax.nn.sigmoid(v)


def _edge0_body(d_ref, nemb_ref, w1_ref, b1_ref, ew_ref, eb_ref, w2_ref,
                b2_ref, sw_ref, sb_ref, ea_out, c_out):
    d = d_ref[...]
    col = lax.broadcasted_iota(jnp.int32, (d.shape[0], 16), 1)
    expo = jnp.where(col < NENC, col,
                     jnp.where(col < 2 * NENC, col - NENC, 0)).astype(F32)
    s = d * jnp.exp(-0.6931471805599453 * expo)
    feats = jnp.where(col < NENC, jnp.sin(s),
                      jnp.where(col < 2 * NENC, jnp.cos(s),
                                jnp.where(col == 2 * NENC, d + 0.0 * s, 0.0)))
    ea = _silu(_silu(jnp.dot(feats, ew_ref[...], preferred_element_type=F32)
                     + eb_ref[...]))
    nemb = nemb_ref[...]
    g0 = (jnp.dot(nemb, w1_ref[0:H, :], preferred_element_type=F32)
          + jnp.dot(nemb, w1_ref[H:2 * H, :], preferred_element_type=F32)
          + b1_ref[...])
    pre1 = g0 + jnp.dot(ea, w1_ref[2 * H:3 * H, :], preferred_element_type=F32)
    m = _silu(jnp.dot(_silu(pre1), w2_ref[...], preferred_element_type=F32)
              + b2_ref[...])
    ea_out[...] = ea + m
    wgt = jax.nn.sigmoid(jnp.dot(m, sw_ref[...], preferred_element_type=F32)
                         + sb_ref[...])
    c_out[...] = m * wgt


def _edge_body(ga_ref, gb_ref, ea_ref, w1_ref, w2_ref, b2_ref, sw_ref,
               sb_ref, ea_out, c_out):
    ea = ea_ref[...]
    pre1 = (ga_ref[...] + gb_ref[...]
            + jnp.dot(ea, w1_ref[2 * H:3 * H, :], preferred_element_type=F32))
    m = _silu(jnp.dot(_silu(pre1), w2_ref[...], preferred_element_type=F32)
              + b2_ref[...])
    ea_out[...] = ea + m
    wgt = jax.nn.sigmoid(jnp.dot(m, sw_ref[...], preferred_element_type=F32)
                         + sb_ref[...])
    c_out[...] = m * wgt


def _full(shape):
    return pl.BlockSpec(shape, lambda i: (0, 0))


def _edge0_call(d2, nemb, w1, b1, ewp, eb, w2, b2, sw, sb):
    grid = E // BE
    return pl.pallas_call(
        _edge0_body,
        grid=(grid,),
        in_specs=[
            pl.BlockSpec((BE, 1), lambda i: (i, 0)),
            _full((1, H)), _full((3 * H, H)), _full((1, H)),
            _full((16, H)), _full((1, H)), _full((H, H)), _full((1, H)),
            _full((H, 1)), _full((1, 1)),
        ],
        out_specs=[pl.BlockSpec((BE, H), lambda i: (i, 0))] * 2,
        out_shape=[jax.ShapeDtypeStruct((E, H), F32)] * 2,
    )(d2, nemb, w1, b1, ewp, eb, w2, b2, sw, sb)


def _edge_call(ga, gb, ea, w1, w2, b2, sw, sb):
    grid = E // BE
    eb_spec = pl.BlockSpec((BE, H), lambda i: (i, 0))
    return pl.pallas_call(
        _edge_body,
        grid=(grid,),
        in_specs=[
            eb_spec, eb_spec, eb_spec,
            _full((3 * H, H)), _full((H, H)), _full((1, H)),
            _full((H, 1)), _full((1, 1)),
        ],
        out_specs=[eb_spec] * 2,
        out_shape=[jax.ShapeDtypeStruct((E, H), F32)] * 2,
    )(ga, gb, ea, w1, w2, b2, sw, sb)


def _node_body(h_is_row, has_ab, a0_ref, a1_ref, h_ref, uw1_ref, ub1_ref,
               uw2_ref, ub2_ref, *rest):
    if has_ab:
        nw1_ref, nb1_ref, h_out, a_out, b_out = rest
    else:
        h_out, = rest
    h = h_ref[...]
    z = a0_ref[...] + a1_ref[...] + h
    hn = (jnp.dot(_silu(jnp.dot(z, uw1_ref[...], preferred_element_type=F32)
                        + ub1_ref[...]),
                  uw2_ref[...], preferred_element_type=F32) + ub2_ref[...])
    h2 = hn + h
    h_out[...] = h2
    if has_ab:
        a_out[...] = jnp.dot(h2, nw1_ref[0:H, :], preferred_element_type=F32)
        b_out[...] = (jnp.dot(h2, nw1_ref[H:2 * H, :],
                              preferred_element_type=F32) + nb1_ref[...])


def _node_call(a0, a1, h, uw1, ub1, uw2, ub2, nw1=None, nb1=None):
    grid = N_PAD // BN
    nb_spec = pl.BlockSpec((BN, H), lambda i: (i, 0))
    h_is_row = (h.shape[0] == 1)
    h_spec = _full((1, H)) if h_is_row else nb_spec
    has_ab = nw1 is not None
    in_specs = [nb_spec, nb_spec, h_spec, _full((H, H)), _full((1, H)),
                _full((H, H)), _full((1, H))]
    args = [a0, a1, h, uw1, ub1, uw2, ub2]
    n_out = 1
    if has_ab:
        in_specs += [_full((2 * H, H)), _full((1, H))]
        args += [nw1, nb1]
        n_out = 3
    return pl.pallas_call(
        functools.partial(_node_body, h_is_row, has_ab),
        grid=(grid,),
        in_specs=in_specs,
        out_specs=[nb_spec] * n_out,
        out_shape=[jax.ShapeDtypeStruct((N_PAD, H), F32)] * n_out,
    )(*args)


def _seg_body(h_ref, b_ref, nw1_ref, nb1_ref, nw2_ref, nb2_ref, s_out,
              cnt_out):
    i = pl.program_id(0)

    @pl.when(i == 0)
    def _():
        s_out[...] = jnp.zeros_like(s_out)
        cnt_out[...] = jnp.zeros_like(cnt_out)

    y = (jnp.dot(_silu(jnp.dot(h_ref[...], nw1_ref[...],
                               preferred_element_type=F32) + nb1_ref[...]),
                 nw2_ref[...], preferred_element_type=F32) + nb2_ref[...])
    b = b_ref[...]
    col = lax.broadcasted_iota(jnp.int32, (BN, G), 1)
    oh = jnp.where(b == col, 1.0, 0.0).astype(F32)
    dn = (((0,), (0,)), ((), ()))
    s_out[...] += lax.dot_general(oh, y, dn, preferred_element_type=F32)
    ones = jnp.ones((BN, 8), F32)
    cnt_out[...] += lax.dot_general(oh, ones, dn, preferred_element_type=F32)


def _seg_call(h, batch2, nw1, nb1, nw2, nb2):
    grid = N_PAD // BN
    return pl.pallas_call(
        _seg_body,
        grid=(grid,),
        in_specs=[
            pl.BlockSpec((BN, H), lambda i: (i, 0)),
            pl.BlockSpec((BN, 1), lambda i: (i, 0)),
            _full((H, H)), _full((1, H)), _full((H, H)), _full((1, H)),
        ],
        out_specs=[_full((G, H)), _full((G, 8))],
        out_shape=[jax.ShapeDtypeStruct((G, H), F32),
                   jax.ShapeDtypeStruct((G, 8), F32)],
    )(h, batch2, nw1, nb1, nw2, nb2)


def _readout_body(s_ref, cnt_ref, w1_ref, b1_ref, w2_ref, b2_ref, out_ref):
    s = s_ref[...]
    cnt = cnt_ref[:, 0:1]
    mean = s / jnp.maximum(cnt, 1.0)
    pre = (jnp.dot(s, w1_ref[0:H, :], preferred_element_type=F32)
           + jnp.dot(mean, w1_ref[H:2 * H, :], preferred_element_type=F32)
           + b1_ref[...])
    out_ref[...] = (jnp.dot(jnp.maximum(pre, 0.0), w2_ref[...],
                            preferred_element_type=F32) + b2_ref[...])


def _readout_call(s, cnt, w1, b1, w2, b2):
    return pl.pallas_call(
        _readout_body,
        out_shape=jax.ShapeDtypeStruct((G, TDIM), F32),
    )(s, cnt, w1, b1, w2, b2)


def _gather_body(a_hbm, b_hbm, src_hbm, dst_hbm, ga_hbm, gb_hbm,
                 idx_s, idx_d, rows_a, rows_b, idx_st, idx_dt, rows_at,
                 rows_bt, sem):
    c = lax.axis_index("c")
    s = lax.axis_index("s")
    base0 = (c * NS + s) * E_PER_W

    def chunk(i, carry):
        base = base0 + i * EC
        pltpu.sync_copy(src_hbm.at[pl.ds(base, EC)], idx_s)
        pltpu.sync_copy(dst_hbm.at[pl.ds(base, EC)], idx_d)
        pltpu.async_copy(a_hbm.at[idx_s], rows_a, sem).wait()
        pltpu.async_copy(b_hbm.at[idx_d], rows_b, sem).wait()
        pltpu.sync_copy(rows_a, ga_hbm.at[pl.ds(base, EC)])
        pltpu.sync_copy(rows_b, gb_hbm.at[pl.ds(base, EC)])
        return carry

    lax.fori_loop(0, NFULL, chunk, 0)
    base = base0 + NFULL * EC
    pltpu.sync_copy(src_hbm.at[pl.ds(base, TAIL)], idx_st)
    pltpu.sync_copy(dst_hbm.at[pl.ds(base, TAIL)], idx_dt)
    pltpu.async_copy(a_hbm.at[idx_st], rows_at, sem).wait()
    pltpu.async_copy(b_hbm.at[idx_dt], rows_bt, sem).wait()
    pltpu.sync_copy(rows_at, ga_hbm.at[pl.ds(base, TAIL)])
    pltpu.sync_copy(rows_bt, gb_hbm.at[pl.ds(base, TAIL)])


_gather_call = pl.kernel(
    _gather_body,
    out_type=[jax.ShapeDtypeStruct((E, H), F32)] * 2,
    mesh=plsc.VectorSubcoreMesh(core_axis_name="c", subcore_axis_name="s"),
    scratch_types=[
        pltpu.VMEM((EC,), jnp.int32), pltpu.VMEM((EC,), jnp.int32),
        pltpu.VMEM((EC, H), F32), pltpu.VMEM((EC, H), F32),
        pltpu.VMEM((TAIL,), jnp.int32), pltpu.VMEM((TAIL,), jnp.int32),
        pltpu.VMEM((TAIL, H), F32), pltpu.VMEM((TAIL, H), F32),
        pltpu.SemaphoreType.DMA,
    ],
)


def _scatter_body(c_hbm, dst_hbm, out_hbm, idx_v, rows_v, idx_t, rows_t,
                  zero_v, agg_sh):
    cc = lax.axis_index("c")
    s = lax.axis_index("s")

    def zstore(k, carry):
        zero_v[k // 8, pl.ds((k % 8) * 16, 16)] = jnp.zeros((16,), F32)
        return carry

    lax.fori_loop(0, EC * 8, zstore, 0)
    row0 = s * ROWS_PER_TILE
    for k in range(ROWS_PER_TILE // EC):
        pltpu.sync_copy(zero_v, agg_sh.at[pl.ds(row0 + k * EC, EC)])
    plsc.subcore_barrier()

    base0 = (cc * NS + s) * E_PER_W

    def chunk(i, carry):
        base = base0 + i * EC
        pltpu.sync_copy(dst_hbm.at[pl.ds(base, EC)], idx_v)
        pltpu.sync_copy(c_hbm.at[pl.ds(base, EC)], rows_v)
        pltpu.sync_copy(rows_v, agg_sh.at[idx_v], add=True)
        return carry

    lax.fori_loop(0, NFULL, chunk, 0)
    base = base0 + NFULL * EC
    pltpu.sync_copy(dst_hbm.at[pl.ds(base, TAIL)], idx_t)
    pltpu.sync_copy(c_hbm.at[pl.ds(base, TAIL)], rows_t)
    pltpu.sync_copy(rows_t, agg_sh.at[idx_t], add=True)
    plsc.subcore_barrier()
    pltpu.sync_copy(agg_sh.at[pl.ds(row0, ROWS_PER_TILE)],
                    out_hbm.at[cc, pl.ds(row0, ROWS_PER_TILE)])


_scatter_call = pl.kernel(
    _scatter_body,
    out_type=jax.ShapeDtypeStruct((NC, N_PAD, H), F32),
    mesh=plsc.VectorSubcoreMesh(core_axis_name="c", subcore_axis_name="s"),
    scratch_types=[
        pltpu.VMEM((EC,), jnp.int32), pltpu.VMEM((EC, H), F32),
        pltpu.VMEM((TAIL,), jnp.int32), pltpu.VMEM((TAIL, H), F32),
        pltpu.VMEM((EC, H), F32),
        pltpu.VMEM_SHARED((N_PAD, H), F32),
    ],
)


def kernel(x, edge_index, edge_attr, batch, node_embedding, edge_W, edge_b,
           msg_W1, msg_b1, msg_W2, msg_b2, soft_W, soft_b, upd_W1, upd_b1,
           upd_W2, upd_b2, nout_W1, nout_b1, nout_W2, nout_b2, ro_W1, ro_b1,
           ro_W2, ro_b2):
    src = edge_index[0].astype(jnp.int32)
    dst = edge_index[1].astype(jnp.int32)
    d2 = edge_attr.reshape(E, 1)
    nemb = node_embedding.reshape(1, H)
    ewp = jnp.zeros((16, H), F32).at[: 2 * NENC + 1].set(edge_W)
    batch2 = jnp.concatenate(
        [batch.astype(jnp.int32), jnp.full((N_PAD - N,), G, jnp.int32)]
    ).reshape(N_PAD, 1)
    row = lambda v: v.reshape(1, -1)

    ea, c = _edge0_call(d2, nemb, msg_W1[0], row(msg_b1[0]), ewp,
                        row(edge_b), msg_W2[0], row(msg_b2[0]), soft_W[0],
                        soft_b[0].reshape(1, 1))
    agg = _scatter_call(c, dst)
    h, A, B = _node_call(agg[0], agg[1], nemb, upd_W1[0], row(upd_b1[0]),
                         upd_W2[0], row(upd_b2[0]), msg_W1[1],
                         row(msg_b1[1]))
    for l in range(1, DEPTH):
        ga, gb = _gather_call(A, B, src, dst)
        ea, c = _edge_call(ga, gb, ea, msg_W1[l], msg_W2[l], row(msg_b2[l]),
                           soft_W[l], soft_b[l].reshape(1, 1))
        agg = _scatter_call(c, dst)
        if l < DEPTH - 1:
            h, A, B = _node_call(agg[0], agg[1], h, upd_W1[l],
                                 row(upd_b1[l]), upd_W2[l], row(upd_b2[l]),
                                 msg_W1[l + 1], row(msg_b1[l + 1]))
        else:
            (h,) = _node_call(agg[0], agg[1], h, upd_W1[l], row(upd_b1[l]),
                              upd_W2[l], row(upd_b2[l]))
    s, cnt = _seg_call(h, batch2, nout_W1, row(nout_b1), nout_W2,
                       row(nout_b2))
    return _readout_call(s, cnt, ro_W1, row(ro_b1), ro_W2, row(ro_b2))


# speedup vs baseline: 3.3206x; 1.2180x over previous
import functools

import jax
import jax.numpy as jnp
import numpy as np
from jax import lax
from jax.experimental import pallas as pl
from jax.experimental.pallas import tpu as pltpu
from jax.experimental.pallas import tpu_sc as plsc

F32 = jnp.float32

N = 10000
E = 320000
H = 128
G = 512
TDIM = 32
DEPTH = 4
NENC = 4

N_PAD = 10240
BN = 2048
BE = 3200

NC, NS = 2, 16
NW = NC * NS
E_PER_W = E // NW
EC = 128
NFULL = E_PER_W // EC
TAIL = E_PER_W - NFULL * EC
ROWS_PER_TILE = N_PAD // NS


def _silu(v):
    return v * jax.nn.sigmoid(v)


def _edge0_body(d_ref, nemb_ref, w1_ref, b1_ref, ew_ref, eb_ref, w2_ref,
                b2_ref, sw_ref, sb_ref, ea_out, c_out):
    d = d_ref[...]
    col = lax.broadcasted_iota(jnp.int32, (d.shape[0], 16), 1)
    expo = jnp.where(col < NENC, col,
                     jnp.where(col < 2 * NENC, col - NENC, 0)).astype(F32)
    s = d * jnp.exp(-0.6931471805599453 * expo)
    feats = jnp.where(col < NENC, jnp.sin(s),
                      jnp.where(col < 2 * NENC, jnp.cos(s),
                                jnp.where(col == 2 * NENC, d + 0.0 * s, 0.0)))
    ea = _silu(_silu(jnp.dot(feats, ew_ref[...], preferred_element_type=F32)
                     + eb_ref[...]))
    nemb = nemb_ref[...]
    g0 = (jnp.dot(nemb, w1_ref[0:H, :], preferred_element_type=F32)
          + jnp.dot(nemb, w1_ref[H:2 * H, :], preferred_element_type=F32)
          + b1_ref[...])
    pre1 = g0 + jnp.dot(ea, w1_ref[2 * H:3 * H, :], preferred_element_type=F32)
    m = _silu(jnp.dot(_silu(pre1), w2_ref[...], preferred_element_type=F32)
              + b2_ref[...])
    ea_out[...] = ea + m
    wgt = jax.nn.sigmoid(jnp.dot(m, sw_ref[...], preferred_element_type=F32)
                         + sb_ref[...])
    c_out[...] = m * wgt


def _edge_body(ga_ref, gb_ref, ea_ref, w1_ref, w2_ref, b2_ref, sw_ref,
               sb_ref, ea_out, c_out):
    ea = ea_ref[...]
    pre1 = (ga_ref[...] + gb_ref[...]
            + jnp.dot(ea, w1_ref[2 * H:3 * H, :], preferred_element_type=F32))
    m = _silu(jnp.dot(_silu(pre1), w2_ref[...], preferred_element_type=F32)
              + b2_ref[...])
    ea_out[...] = ea + m
    wgt = jax.nn.sigmoid(jnp.dot(m, sw_ref[...], preferred_element_type=F32)
                         + sb_ref[...])
    c_out[...] = m * wgt


def _full(shape):
    return pl.BlockSpec(shape, lambda i: (0, 0))


def _edge0_call(d2, nemb, w1, b1, ewp, eb, w2, b2, sw, sb):
    grid = E // BE
    return pl.pallas_call(
        _edge0_body,
        grid=(grid,),
        in_specs=[
            pl.BlockSpec((BE, 1), lambda i: (i, 0)),
            _full((1, H)), _full((3 * H, H)), _full((1, H)),
            _full((16, H)), _full((1, H)), _full((H, H)), _full((1, H)),
            _full((H, 1)), _full((1, 1)),
        ],
        out_specs=[pl.BlockSpec((BE, H), lambda i: (i, 0))] * 2,
        out_shape=[jax.ShapeDtypeStruct((E, H), F32)] * 2,
    )(d2, nemb, w1, b1, ewp, eb, w2, b2, sw, sb)


def _edge_call(ga, gb, ea, w1, w2, b2, sw, sb):
    grid = E // BE
    eb_spec = pl.BlockSpec((BE, H), lambda i: (i, 0))
    return pl.pallas_call(
        _edge_body,
        grid=(grid,),
        in_specs=[
            eb_spec, eb_spec, eb_spec,
            _full((3 * H, H)), _full((H, H)), _full((1, H)),
            _full((H, 1)), _full((1, 1)),
        ],
        out_specs=[eb_spec] * 2,
        out_shape=[jax.ShapeDtypeStruct((E, H), F32)] * 2,
    )(ga, gb, ea, w1, w2, b2, sw, sb)


def _node_body(h_is_row, has_ab, a0_ref, a1_ref, h_ref, uw1_ref, ub1_ref,
               uw2_ref, ub2_ref, *rest):
    if has_ab:
        nw1_ref, nb1_ref, h_out, a_out, b_out = rest
    else:
        h_out, = rest
    h = h_ref[...]
    z = a0_ref[...] + a1_ref[...] + h
    hn = (jnp.dot(_silu(jnp.dot(z, uw1_ref[...], preferred_element_type=F32)
                        + ub1_ref[...]),
                  uw2_ref[...], preferred_element_type=F32) + ub2_ref[...])
    h2 = hn + h
    h_out[...] = h2
    if has_ab:
        a_out[...] = jnp.dot(h2, nw1_ref[0:H, :], preferred_element_type=F32)
        b_out[...] = (jnp.dot(h2, nw1_ref[H:2 * H, :],
                              preferred_element_type=F32) + nb1_ref[...])


def _node_call(a0, a1, h, uw1, ub1, uw2, ub2, nw1=None, nb1=None):
    grid = N_PAD // BN
    nb_spec = pl.BlockSpec((BN, H), lambda i: (i, 0))
    h_is_row = (h.shape[0] == 1)
    h_spec = _full((1, H)) if h_is_row else nb_spec
    has_ab = nw1 is not None
    in_specs = [nb_spec, nb_spec, h_spec, _full((H, H)), _full((1, H)),
                _full((H, H)), _full((1, H))]
    args = [a0, a1, h, uw1, ub1, uw2, ub2]
    n_out = 1
    if has_ab:
        in_specs += [_full((2 * H, H)), _full((1, H))]
        args += [nw1, nb1]
        n_out = 3
    return pl.pallas_call(
        functools.partial(_node_body, h_is_row, has_ab),
        grid=(grid,),
        in_specs=in_specs,
        out_specs=[nb_spec] * n_out,
        out_shape=[jax.ShapeDtypeStruct((N_PAD, H), F32)] * n_out,
    )(*args)


def _seg_body(h_ref, b_ref, nw1_ref, nb1_ref, nw2_ref, nb2_ref, s_out,
              cnt_out):
    i = pl.program_id(0)

    @pl.when(i == 0)
    def _():
        s_out[...] = jnp.zeros_like(s_out)
        cnt_out[...] = jnp.zeros_like(cnt_out)

    y = (jnp.dot(_silu(jnp.dot(h_ref[...], nw1_ref[...],
                               preferred_element_type=F32) + nb1_ref[...]),
                 nw2_ref[...], preferred_element_type=F32) + nb2_ref[...])
    b = b_ref[...]
    col = lax.broadcasted_iota(jnp.int32, (BN, G), 1)
    oh = jnp.where(b == col, 1.0, 0.0).astype(F32)
    dn = (((0,), (0,)), ((), ()))
    s_out[...] += lax.dot_general(oh, y, dn, preferred_element_type=F32)
    ones = jnp.ones((BN, 8), F32)
    cnt_out[...] += lax.dot_general(oh, ones, dn, preferred_element_type=F32)


def _seg_call(h, batch2, nw1, nb1, nw2, nb2):
    grid = N_PAD // BN
    return pl.pallas_call(
        _seg_body,
        grid=(grid,),
        in_specs=[
            pl.BlockSpec((BN, H), lambda i: (i, 0)),
            pl.BlockSpec((BN, 1), lambda i: (i, 0)),
            _full((H, H)), _full((1, H)), _full((H, H)), _full((1, H)),
        ],
        out_specs=[_full((G, H)), _full((G, 8))],
        out_shape=[jax.ShapeDtypeStruct((G, H), F32),
                   jax.ShapeDtypeStruct((G, 8), F32)],
    )(h, batch2, nw1, nb1, nw2, nb2)


def _readout_body(s_ref, cnt_ref, w1_ref, b1_ref, w2_ref, b2_ref, out_ref):
    s = s_ref[...]
    cnt = cnt_ref[:, 0:1]
    mean = s / jnp.maximum(cnt, 1.0)
    pre = (jnp.dot(s, w1_ref[0:H, :], preferred_element_type=F32)
           + jnp.dot(mean, w1_ref[H:2 * H, :], preferred_element_type=F32)
           + b1_ref[...])
    out_ref[...] = (jnp.dot(jnp.maximum(pre, 0.0), w2_ref[...],
                            preferred_element_type=F32) + b2_ref[...])


def _readout_call(s, cnt, w1, b1, w2, b2):
    return pl.pallas_call(
        _readout_body,
        out_shape=jax.ShapeDtypeStruct((G, TDIM), F32),
    )(s, cnt, w1, b1, w2, b2)


def _gather_body(a_hbm, b_hbm, src_hbm, dst_hbm, ga_hbm, gb_hbm,
                 idx_s, idx_d, rows_a, rows_b, idx_st, idx_dt, rows_at,
                 rows_bt, sem_i0, sem_i1, sem_g0, sem_g1, sem_o):
    c = lax.axis_index("c")
    s = lax.axis_index("s")
    base0 = (c * NS + s) * E_PER_W
    sem_i = (sem_i0, sem_i1)
    sem_g = (sem_g0, sem_g1)

    def pair(j, carry):
        base = [base0 + (2 * j + b) * EC for b in (0, 1)]
        cps = []
        for b in (0, 1):
            cps.append(pltpu.async_copy(src_hbm.at[pl.ds(base[b], EC)],
                                        idx_s.at[b], sem_i[b]))
            cps.append(pltpu.async_copy(dst_hbm.at[pl.ds(base[b], EC)],
                                        idx_d.at[b], sem_i[b]))
        gs = []
        for b in (0, 1):
            cps[2 * b].wait()
            cps[2 * b + 1].wait()
            gs.append(pltpu.async_copy(a_hbm.at[idx_s.at[b]], rows_a.at[b],
                                       sem_g[b]))
            gs.append(pltpu.async_copy(b_hbm.at[idx_d.at[b]], rows_b.at[b],
                                       sem_g[b]))
        os_ = []
        for b in (0, 1):
            gs[2 * b].wait()
            gs[2 * b + 1].wait()
            os_.append(pltpu.async_copy(rows_a.at[b],
                                        ga_hbm.at[pl.ds(base[b], EC)], sem_o))
            os_.append(pltpu.async_copy(rows_b.at[b],
                                        gb_hbm.at[pl.ds(base[b], EC)], sem_o))
        for o in os_:
            o.wait()
        return carry

    lax.fori_loop(0, NFULL // 2, pair, 0)
    base = base0 + NFULL * EC
    pltpu.sync_copy(src_hbm.at[pl.ds(base, TAIL)], idx_st)
    pltpu.sync_copy(dst_hbm.at[pl.ds(base, TAIL)], idx_dt)
    pltpu.async_copy(a_hbm.at[idx_st], rows_at, sem_g0).wait()
    pltpu.async_copy(b_hbm.at[idx_dt], rows_bt, sem_g1).wait()
    pltpu.sync_copy(rows_at, ga_hbm.at[pl.ds(base, TAIL)])
    pltpu.sync_copy(rows_bt, gb_hbm.at[pl.ds(base, TAIL)])


_gather_call = pl.kernel(
    _gather_body,
    out_type=[jax.ShapeDtypeStruct((E, H), F32)] * 2,
    mesh=plsc.VectorSubcoreMesh(core_axis_name="c", subcore_axis_name="s"),
    scratch_types=[
        pltpu.VMEM((2, EC), jnp.int32), pltpu.VMEM((2, EC), jnp.int32),
        pltpu.VMEM((2, EC, H), F32), pltpu.VMEM((2, EC, H), F32),
        pltpu.VMEM((TAIL,), jnp.int32), pltpu.VMEM((TAIL,), jnp.int32),
        pltpu.VMEM((TAIL, H), F32), pltpu.VMEM((TAIL, H), F32),
        pltpu.SemaphoreType.DMA, pltpu.SemaphoreType.DMA,
        pltpu.SemaphoreType.DMA, pltpu.SemaphoreType.DMA,
        pltpu.SemaphoreType.DMA,
    ],
)


def _scatter_body(c_hbm, dst_hbm, out_hbm, idx_v, rows_v, idx_t, rows_t,
                  agg_sh, sem_i0, sem_i1, sem_s):
    cc = lax.axis_index("c")
    s = lax.axis_index("s")

    def zstore(k, carry):
        rows_v[0, k // 8, pl.ds((k % 8) * 16, 16)] = jnp.zeros((16,), F32)
        return carry

    lax.fori_loop(0, EC * 8, zstore, 0)
    row0 = s * ROWS_PER_TILE
    for k in range(ROWS_PER_TILE // EC):
        pltpu.sync_copy(rows_v.at[0], agg_sh.at[pl.ds(row0 + k * EC, EC)])
    plsc.subcore_barrier()

    base0 = (cc * NS + s) * E_PER_W
    sem_i = (sem_i0, sem_i1)

    def pair(j, carry):
        base = [base0 + (2 * j + b) * EC for b in (0, 1)]
        cps = []
        for b in (0, 1):
            cps.append(pltpu.async_copy(dst_hbm.at[pl.ds(base[b], EC)],
                                        idx_v.at[b], sem_i[b]))
            cps.append(pltpu.async_copy(c_hbm.at[pl.ds(base[b], EC)],
                                        rows_v.at[b], sem_i[b]))
        ss = []
        for b in (0, 1):
            cps[2 * b].wait()
            cps[2 * b + 1].wait()
            ss.append(pltpu.async_copy(rows_v.at[b], agg_sh.at[idx_v.at[b]],
                                       sem_s, add=True))
        for s_ in ss:
            s_.wait()
        return carry

    lax.fori_loop(0, NFULL // 2, pair, 0)
    base = base0 + NFULL * EC
    pltpu.sync_copy(dst_hbm.at[pl.ds(base, TAIL)], idx_t)
    pltpu.sync_copy(c_hbm.at[pl.ds(base, TAIL)], rows_t)
    pltpu.sync_copy(rows_t, agg_sh.at[idx_t], add=True)
    plsc.subcore_barrier()
    pltpu.sync_copy(agg_sh.at[pl.ds(row0, ROWS_PER_TILE)],
                    out_hbm.at[cc, pl.ds(row0, ROWS_PER_TILE)])


_scatter_call = pl.kernel(
    _scatter_body,
    out_type=jax.ShapeDtypeStruct((NC, N_PAD, H), F32),
    mesh=plsc.VectorSubcoreMesh(core_axis_name="c", subcore_axis_name="s"),
    scratch_types=[
        pltpu.VMEM((2, EC), jnp.int32), pltpu.VMEM((2, EC, H), F32),
        pltpu.VMEM((TAIL,), jnp.int32), pltpu.VMEM((TAIL, H), F32),
        pltpu.VMEM_SHARED((N_PAD, H), F32),
        pltpu.SemaphoreType.DMA, pltpu.SemaphoreType.DMA,
        pltpu.SemaphoreType.DMA,
    ],
)


def kernel(x, edge_index, edge_attr, batch, node_embedding, edge_W, edge_b,
           msg_W1, msg_b1, msg_W2, msg_b2, soft_W, soft_b, upd_W1, upd_b1,
           upd_W2, upd_b2, nout_W1, nout_b1, nout_W2, nout_b2, ro_W1, ro_b1,
           ro_W2, ro_b2):
    src = edge_index[0].astype(jnp.int32)
    dst = edge_index[1].astype(jnp.int32)
    d2 = edge_attr.reshape(E, 1)
    nemb = node_embedding.reshape(1, H)
    ewp = jnp.zeros((16, H), F32).at[: 2 * NENC + 1].set(edge_W)
    batch2 = jnp.concatenate(
        [batch.astype(jnp.int32), jnp.full((N_PAD - N,), G, jnp.int32)]
    ).reshape(N_PAD, 1)
    row = lambda v: v.reshape(1, -1)

    ea, c = _edge0_call(d2, nemb, msg_W1[0], row(msg_b1[0]), ewp,
                        row(edge_b), msg_W2[0], row(msg_b2[0]), soft_W[0],
                        soft_b[0].reshape(1, 1))
    agg = _scatter_call(c, dst)
    h, A, B = _node_call(agg[0], agg[1], nemb, upd_W1[0], row(upd_b1[0]),
                         upd_W2[0], row(upd_b2[0]), msg_W1[1],
                         row(msg_b1[1]))
    for l in range(1, DEPTH):
        ga, gb = _gather_call(A, B, src, dst)
        ea, c = _edge_call(ga, gb, ea, msg_W1[l], msg_W2[l], row(msg_b2[l]),
                           soft_W[l], soft_b[l].reshape(1, 1))
        agg = _scatter_call(c, dst)
        if l < DEPTH - 1:
            h, A, B = _node_call(agg[0], agg[1], h, upd_W1[l],
                                 row(upd_b1[l]), upd_W2[l], row(upd_b2[l]),
                                 msg_W1[l + 1], row(msg_b1[l + 1]))
        else:
            (h,) = _node_call(agg[0], agg[1], h, upd_W1[l], row(upd_b1[l]),
                              upd_W2[l], row(upd_b2[l]))
    s, cnt = _seg_call(h, batch2, nout_W1, row(nout_b1), nout_W2,
                       row(nout_b2))
    return _readout_call(s, cnt, ro_W1, row(ro_b1), ro_W2, row(ro_b2))


# speedup vs baseline: 4.1699x; 1.2558x over previous
import functools

import jax
import jax.numpy as jnp
import numpy as np
from jax import lax
from jax.experimental import pallas as pl
from jax.experimental.pallas import tpu as pltpu
from jax.experimental.pallas import tpu_sc as plsc

F32 = jnp.float32

N = 10000
E = 320000
H = 128
G = 512
TDIM = 32
DEPTH = 4
NENC = 4

N_PAD = 10240
BN = 2048
BE = 3200

NC, NS = 2, 16
NW = NC * NS
E_PER_W = E // NW
EC = 128
NFULL = E_PER_W // EC
TAIL = E_PER_W - NFULL * EC
ROWS_PER_TILE = N_PAD // NS


def _silu(v):
    return v * jax.nn.sigmoid(v)


def _edge0_body(d_ref, nemb_ref, w1_ref, b1_ref, ew_ref, eb_ref, w2_ref,
                b2_ref, sw_ref, sb_ref, ea_out, c_out):
    rowi = lax.broadcasted_iota(jnp.int32, (16, 128), 0)
    expo = jnp.where(rowi < NENC, rowi,
                     jnp.where(rowi < 2 * NENC, rowi - NENC, 0)).astype(F32)
    scale = jnp.exp(-0.6931471805599453 * expo)
    dn = (((0,), (0,)), ((), ()))
    pieces = []
    for r in range(BE // 128):
        drow = d_ref[0, r:r + 1, :]
        s = drow * scale
        feats = jnp.where(
            rowi < NENC, jnp.sin(s),
            jnp.where(rowi < 2 * NENC, jnp.cos(s),
                      jnp.where(rowi == 2 * NENC,
                                jnp.broadcast_to(drow, (16, 128)), 0.0)))
        pieces.append(lax.dot_general(feats, ew_ref[...], dn,
                                      preferred_element_type=F32))
    ea = _silu(_silu(jnp.concatenate(pieces, axis=0) + eb_ref[...]))
    nemb = nemb_ref[...]
    g0 = (jnp.dot(nemb, w1_ref[0:H, :], preferred_element_type=F32)
          + jnp.dot(nemb, w1_ref[H:2 * H, :], preferred_element_type=F32)
          + b1_ref[...])
    pre1 = g0 + jnp.dot(ea, w1_ref[2 * H:3 * H, :], preferred_element_type=F32)
    m = _silu(jnp.dot(_silu(pre1), w2_ref[...], preferred_element_type=F32)
              + b2_ref[...])
    ea_out[...] = ea + m
    wgt = jax.nn.sigmoid(jnp.dot(m, sw_ref[...], preferred_element_type=F32)
                         + sb_ref[...])
    c_out[...] = m * wgt


def _edge_body(ga_ref, gb_ref, ea_ref, w1_ref, w2_ref, b2_ref, sw_ref,
               sb_ref, ea_out, c_out):
    ea = ea_ref[...]
    pre1 = (ga_ref[...] + gb_ref[...]
            + jnp.dot(ea, w1_ref[2 * H:3 * H, :], preferred_element_type=F32))
    m = _silu(jnp.dot(_silu(pre1), w2_ref[...], preferred_element_type=F32)
              + b2_ref[...])
    ea_out[...] = ea + m
    wgt = jax.nn.sigmoid(jnp.dot(m, sw_ref[...], preferred_element_type=F32)
                         + sb_ref[...])
    c_out[...] = m * wgt


def _full(shape):
    return pl.BlockSpec(shape, lambda i: (0, 0))


def _edge0_call(d2, nemb, w1, b1, ewp, eb, w2, b2, sw, sb):
    grid = E // BE
    return pl.pallas_call(
        _edge0_body,
        grid=(grid,),
        in_specs=[
            pl.BlockSpec((1, BE // 128, 128), lambda i: (i, 0, 0)),
            _full((1, H)), _full((3 * H, H)), _full((1, H)),
            _full((16, H)), _full((1, H)), _full((H, H)), _full((1, H)),
            _full((H, 1)), _full((1, 1)),
        ],
        out_specs=[pl.BlockSpec((BE, H), lambda i: (i, 0))] * 2,
        out_shape=[jax.ShapeDtypeStruct((E, H), F32)] * 2,
    )(d2, nemb, w1, b1, ewp, eb, w2, b2, sw, sb)


def _edge_call(ga, gb, ea, w1, w2, b2, sw, sb):
    grid = E // BE
    eb_spec = pl.BlockSpec((BE, H), lambda i: (i, 0))
    return pl.pallas_call(
        _edge_body,
        grid=(grid,),
        in_specs=[
            eb_spec, eb_spec, eb_spec,
            _full((3 * H, H)), _full((H, H)), _full((1, H)),
            _full((H, 1)), _full((1, 1)),
        ],
        out_specs=[eb_spec] * 2,
        out_shape=[jax.ShapeDtypeStruct((E, H), F32)] * 2,
    )(ga, gb, ea, w1, w2, b2, sw, sb)


def _node_body(h_is_row, has_ab, a0_ref, a1_ref, h_ref, uw1_ref, ub1_ref,
               uw2_ref, ub2_ref, *rest):
    if has_ab:
        nw1_ref, nb1_ref, h_out, a_out, b_out = rest
    else:
        h_out, = rest
    h = h_ref[...]
    z = a0_ref[...] + a1_ref[...] + h
    hn = (jnp.dot(_silu(jnp.dot(z, uw1_ref[...], preferred_element_type=F32)
                        + ub1_ref[...]),
                  uw2_ref[...], preferred_element_type=F32) + ub2_ref[...])
    h2 = hn + h
    h_out[...] = h2
    if has_ab:
        a_out[...] = jnp.dot(h2, nw1_ref[0:H, :], preferred_element_type=F32)
        b_out[...] = (jnp.dot(h2, nw1_ref[H:2 * H, :],
                              preferred_element_type=F32) + nb1_ref[...])


def _node_call(a0, a1, h, uw1, ub1, uw2, ub2, nw1=None, nb1=None):
    grid = N_PAD // BN
    nb_spec = pl.BlockSpec((BN, H), lambda i: (i, 0))
    h_is_row = (h.shape[0] == 1)
    h_spec = _full((1, H)) if h_is_row else nb_spec
    has_ab = nw1 is not None
    in_specs = [nb_spec, nb_spec, h_spec, _full((H, H)), _full((1, H)),
                _full((H, H)), _full((1, H))]
    args = [a0, a1, h, uw1, ub1, uw2, ub2]
    n_out = 1
    if has_ab:
        in_specs += [_full((2 * H, H)), _full((1, H))]
        args += [nw1, nb1]
        n_out = 3
    return pl.pallas_call(
        functools.partial(_node_body, h_is_row, has_ab),
        grid=(grid,),
        in_specs=in_specs,
        out_specs=[nb_spec] * n_out,
        out_shape=[jax.ShapeDtypeStruct((N_PAD, H), F32)] * n_out,
    )(*args)


def _seg_body(h_ref, b_ref, nw1_ref, nb1_ref, nw2_ref, nb2_ref, s_out,
              cnt_out):
    i = pl.program_id(0)

    @pl.when(i == 0)
    def _():
        s_out[...] = jnp.zeros_like(s_out)
        cnt_out[...] = jnp.zeros_like(cnt_out)

    y = (jnp.dot(_silu(jnp.dot(h_ref[...], nw1_ref[...],
                               preferred_element_type=F32) + nb1_ref[...]),
                 nw2_ref[...], preferred_element_type=F32) + nb2_ref[...])
    b = b_ref[...]
    col = lax.broadcasted_iota(jnp.int32, (BN, G), 1)
    oh = jnp.where(b == col, 1.0, 0.0).astype(F32)
    dn = (((0,), (0,)), ((), ()))
    s_out[...] += lax.dot_general(oh, y, dn, preferred_element_type=F32)
    ones = jnp.ones((BN, 8), F32)
    cnt_out[...] += lax.dot_general(oh, ones, dn, preferred_element_type=F32)


def _seg_call(h, batch2, nw1, nb1, nw2, nb2):
    grid = N_PAD // BN
    return pl.pallas_call(
        _seg_body,
        grid=(grid,),
        in_specs=[
            pl.BlockSpec((BN, H), lambda i: (i, 0)),
            pl.BlockSpec((BN, 1), lambda i: (i, 0)),
            _full((H, H)), _full((1, H)), _full((H, H)), _full((1, H)),
        ],
        out_specs=[_full((G, H)), _full((G, 8))],
        out_shape=[jax.ShapeDtypeStruct((G, H), F32),
                   jax.ShapeDtypeStruct((G, 8), F32)],
    )(h, batch2, nw1, nb1, nw2, nb2)


def _readout_body(s_ref, cnt_ref, w1_ref, b1_ref, w2_ref, b2_ref, out_ref):
    s = s_ref[...]
    cnt = cnt_ref[:, 0:1]
    mean = s / jnp.maximum(cnt, 1.0)
    pre = (jnp.dot(s, w1_ref[0:H, :], preferred_element_type=F32)
           + jnp.dot(mean, w1_ref[H:2 * H, :], preferred_element_type=F32)
           + b1_ref[...])
    out_ref[...] = (jnp.dot(jnp.maximum(pre, 0.0), w2_ref[...],
                            preferred_element_type=F32) + b2_ref[...])


def _readout_call(s, cnt, w1, b1, w2, b2):
    return pl.pallas_call(
        _readout_body,
        out_shape=jax.ShapeDtypeStruct((G, TDIM), F32),
    )(s, cnt, w1, b1, w2, b2)


def _gather_body(a_hbm, b_hbm, src_hbm, dst_hbm, ga_hbm, gb_hbm,
                 idx_s, idx_d, rows_a, rows_b, idx_st, idx_dt, rows_at,
                 rows_bt, sem_i0, sem_i1, sem_g0, sem_g1, sem_o):
    c = lax.axis_index("c")
    s = lax.axis_index("s")
    base0 = (c * NS + s) * E_PER_W
    sem_i = (sem_i0, sem_i1)
    sem_g = (sem_g0, sem_g1)

    def pair(j, carry):
        base = [base0 + (2 * j + b) * EC for b in (0, 1)]
        cps = []
        for b in (0, 1):
            cps.append(pltpu.async_copy(src_hbm.at[pl.ds(base[b], EC)],
                                        idx_s.at[b], sem_i[b]))
            cps.append(pltpu.async_copy(dst_hbm.at[pl.ds(base[b], EC)],
                                        idx_d.at[b], sem_i[b]))
        gs = []
        for b in (0, 1):
            cps[2 * b].wait()
            cps[2 * b + 1].wait()
            gs.append(pltpu.async_copy(a_hbm.at[idx_s.at[b]], rows_a.at[b],
                                       sem_g[b]))
            gs.append(pltpu.async_copy(b_hbm.at[idx_d.at[b]], rows_b.at[b],
                                       sem_g[b]))
        os_ = []
        for b in (0, 1):
            gs[2 * b].wait()
            gs[2 * b + 1].wait()
            os_.append(pltpu.async_copy(rows_a.at[b],
                                        ga_hbm.at[pl.ds(base[b], EC)], sem_o))
            os_.append(pltpu.async_copy(rows_b.at[b],
                                        gb_hbm.at[pl.ds(base[b], EC)], sem_o))
        for o in os_:
            o.wait()
        return carry

    lax.fori_loop(0, NFULL // 2, pair, 0)
    base = base0 + NFULL * EC
    pltpu.sync_copy(src_hbm.at[pl.ds(base, TAIL)], idx_st)
    pltpu.sync_copy(dst_hbm.at[pl.ds(base, TAIL)], idx_dt)
    pltpu.async_copy(a_hbm.at[idx_st], rows_at, sem_g0).wait()
    pltpu.async_copy(b_hbm.at[idx_dt], rows_bt, sem_g1).wait()
    pltpu.sync_copy(rows_at, ga_hbm.at[pl.ds(base, TAIL)])
    pltpu.sync_copy(rows_bt, gb_hbm.at[pl.ds(base, TAIL)])


_gather_call = pl.kernel(
    _gather_body,
    out_type=[jax.ShapeDtypeStruct((E, H), F32)] * 2,
    mesh=plsc.VectorSubcoreMesh(core_axis_name="c", subcore_axis_name="s"),
    scratch_types=[
        pltpu.VMEM((2, EC), jnp.int32), pltpu.VMEM((2, EC), jnp.int32),
        pltpu.VMEM((2, EC, H), F32), pltpu.VMEM((2, EC, H), F32),
        pltpu.VMEM((TAIL,), jnp.int32), pltpu.VMEM((TAIL,), jnp.int32),
        pltpu.VMEM((TAIL, H), F32), pltpu.VMEM((TAIL, H), F32),
        pltpu.SemaphoreType.DMA, pltpu.SemaphoreType.DMA,
        pltpu.SemaphoreType.DMA, pltpu.SemaphoreType.DMA,
        pltpu.SemaphoreType.DMA,
    ],
)


def _scatter_body(c_hbm, dst_hbm, out_hbm, idx_v, rows_v, idx_t, rows_t,
                  agg_sh, sem_i0, sem_i1, sem_s):
    cc = lax.axis_index("c")
    s = lax.axis_index("s")

    def zstore(k, carry):
        rows_v[0, k // 8, pl.ds((k % 8) * 16, 16)] = jnp.zeros((16,), F32)
        return carry

    lax.fori_loop(0, EC * 8, zstore, 0)
    row0 = s * ROWS_PER_TILE
    for k in range(ROWS_PER_TILE // EC):
        pltpu.sync_copy(rows_v.at[0], agg_sh.at[pl.ds(row0 + k * EC, EC)])
    plsc.subcore_barrier()

    base0 = (cc * NS + s) * E_PER_W
    sem_i = (sem_i0, sem_i1)

    def pair(j, carry):
        base = [base0 + (2 * j + b) * EC for b in (0, 1)]
        cps = []
        for b in (0, 1):
            cps.append(pltpu.async_copy(dst_hbm.at[pl.ds(base[b], EC)],
                                        idx_v.at[b], sem_i[b]))
            cps.append(pltpu.async_copy(c_hbm.at[pl.ds(base[b], EC)],
                                        rows_v.at[b], sem_i[b]))
        ss = []
        for b in (0, 1):
            cps[2 * b].wait()
            cps[2 * b + 1].wait()
            ss.append(pltpu.async_copy(rows_v.at[b], agg_sh.at[idx_v.at[b]],
                                       sem_s, add=True))
        for s_ in ss:
            s_.wait()
        return carry

    lax.fori_loop(0, NFULL // 2, pair, 0)
    base = base0 + NFULL * EC
    pltpu.sync_copy(dst_hbm.at[pl.ds(base, TAIL)], idx_t)
    pltpu.sync_copy(c_hbm.at[pl.ds(base, TAIL)], rows_t)
    pltpu.sync_copy(rows_t, agg_sh.at[idx_t], add=True)
    plsc.subcore_barrier()
    pltpu.sync_copy(agg_sh.at[pl.ds(row0, ROWS_PER_TILE)],
                    out_hbm.at[cc, pl.ds(row0, ROWS_PER_TILE)])


_scatter_call = pl.kernel(
    _scatter_body,
    out_type=jax.ShapeDtypeStruct((NC, N_PAD, H), F32),
    mesh=plsc.VectorSubcoreMesh(core_axis_name="c", subcore_axis_name="s"),
    scratch_types=[
        pltpu.VMEM((2, EC), jnp.int32), pltpu.VMEM((2, EC, H), F32),
        pltpu.VMEM((TAIL,), jnp.int32), pltpu.VMEM((TAIL, H), F32),
        pltpu.VMEM_SHARED((N_PAD, H), F32),
        pltpu.SemaphoreType.DMA, pltpu.SemaphoreType.DMA,
        pltpu.SemaphoreType.DMA,
    ],
)


def kernel(x, edge_index, edge_attr, batch, node_embedding, edge_W, edge_b,
           msg_W1, msg_b1, msg_W2, msg_b2, soft_W, soft_b, upd_W1, upd_b1,
           upd_W2, upd_b2, nout_W1, nout_b1, nout_W2, nout_b2, ro_W1, ro_b1,
           ro_W2, ro_b2):
    src = edge_index[0].astype(jnp.int32)
    dst = edge_index[1].astype(jnp.int32)
    d2 = edge_attr.reshape(E // BE, BE // 128, 128)
    nemb = node_embedding.reshape(1, H)
    ewp = jnp.zeros((16, H), F32).at[: 2 * NENC + 1].set(edge_W)
    batch2 = jnp.concatenate(
        [batch.astype(jnp.int32), jnp.full((N_PAD - N,), G, jnp.int32)]
    ).reshape(N_PAD, 1)
    row = lambda v: v.reshape(1, -1)

    ea, c = _edge0_call(d2, nemb, msg_W1[0], row(msg_b1[0]), ewp,
                        row(edge_b), msg_W2[0], row(msg_b2[0]), soft_W[0],
                        soft_b[0].reshape(1, 1))
    agg = _scatter_call(c, dst)
    h, A, B = _node_call(agg[0], agg[1], nemb, upd_W1[0], row(upd_b1[0]),
                         upd_W2[0], row(upd_b2[0]), msg_W1[1],
                         row(msg_b1[1]))
    for l in range(1, DEPTH):
        ga, gb = _gather_call(A, B, src, dst)
        ea, c = _edge_call(ga, gb, ea, msg_W1[l], msg_W2[l], row(msg_b2[l]),
                           soft_W[l], soft_b[l].reshape(1, 1))
        agg = _scatter_call(c, dst)
        if l < DEPTH - 1:
            h, A, B = _node_call(agg[0], agg[1], h, upd_W1[l],
                                 row(upd_b1[l]), upd_W2[l], row(upd_b2[l]),
                                 msg_W1[l + 1], row(msg_b1[l + 1]))
        else:
            (h,) = _node_call(agg[0], agg[1], h, upd_W1[l], row(upd_b1[l]),
                              upd_W2[l], row(upd_b2[l]))
    s, cnt = _seg_call(h, batch2, nout_W1, row(nout_b1), nout_W2,
                       row(nout_b2))
    return _readout_call(s, cnt, ro_W1, row(ro_b1), ro_W2, row(ro_b2))
